# baseline (speedup 1.0000x reference)
import functools

import jax
import jax.numpy as jnp
from jax import lax
from jax.experimental import pallas as pl
from jax.experimental.pallas import tpu as pltpu
from jax.experimental.pallas import tpu_sc as plsc

B = 64
L = 1024
D = 768
KEEP = 512
NPREF = 1
LIN = L + NPREF
LOUT = KEEP + NPREF

NC = 2
NS = 16
NW = NC * NS
ROWS_PER_W = B // NW
LANES = 16

CH = 32
NCH = 16
NBUF = 4


RG = 8


def _rank_body(nlane_ref, out_ref):
    kidx = lax.broadcasted_iota(jnp.int32, (L, L), 0)
    jidx = lax.broadcasted_iota(jnp.int32, (L, L), 1)
    tie = kidx < jidx
    for g in range(RG):
        nj = nlane_ref[g]
        nk = jnp.swapaxes(nj, 0, 1)
        lt = nk < nj
        eq = nk == nj
        prec = lt | (eq & tie)
        rank = jnp.sum(prec.astype(jnp.int32), axis=0, keepdims=True)
        out_ref[g] = rank


def _ranks(noise):
    noise3 = noise.reshape(B, 1, L)
    out = pl.pallas_call(
        _rank_body,
        grid=(B // RG,),
        in_specs=[pl.BlockSpec((RG, 1, L), lambda b: (b, 0, 0))],
        out_specs=pl.BlockSpec((RG, 1, L), lambda b: (b, 0, 0)),
        out_shape=jax.ShapeDtypeStruct((B, 1, L), jnp.int32),
    )(noise3)
    return out


def _sc_body(x_hbm, ranks_hbm, out_hbm, rank_v, kp_v, oix_v,
             buf0, buf1, buf2, buf3, gsem0, gsem1, ssem0, ssem1, ssem2,
             ssem3, semp):
    wid = lax.axis_index("s") * NC + lax.axis_index("c")
    lane = jnp.arange(LANES, dtype=jnp.int32)

    @pl.when(wid == 0)
    def _():
        pltpu.async_copy(x_hbm.at[pl.ds(0, B)], out_hbm.at[pl.ds(0, B)],
                         semp).wait()

    for i in range(ROWS_PER_W):
        b = wid * ROWS_PER_W + i
        pltpu.sync_copy(ranks_hbm.at[b], rank_v)
        for c in range(NCH):
            for g in range(CH // LANES):
                oix_v[c, pl.ds(g * LANES, LANES)] = (
                    lane * B + ((1 + c * CH + g * LANES) * B + b))
        for ci in range(L // LANES):
            r = rank_v[0, pl.ds(ci * LANES, LANES)]
            src = lane * B + ((ci * LANES + 1) * B + b)
            plsc.store_scatter(kp_v, [r // CH, r % CH], src, mask=r < KEEP)
        bufs = (buf0, buf1, buf2, buf3)
        gsems = (gsem0, gsem1)
        ssems = (ssem0, ssem1, ssem2, ssem3)
        gd = [None] * NCH
        sd = [None] * NCH
        gd[0] = pltpu.async_copy(x_hbm.at[kp_v.at[0]], bufs[0], gsems[0])
        gd[1] = pltpu.async_copy(x_hbm.at[kp_v.at[1]], bufs[1], gsems[1])
        for c in range(NCH):
            gd[c].wait()
            sd[c] = pltpu.async_copy(bufs[c % NBUF], out_hbm.at[oix_v.at[c]],
                                     ssems[c % NBUF])
            nxt = c + 2
            if nxt < NCH:
                if c >= 2:
                    sd[c - 2].wait()
                gd[nxt] = pltpu.async_copy(x_hbm.at[kp_v.at[nxt]],
                                           bufs[nxt % NBUF],
                                           gsems[nxt % 2])
        for c in range(NCH - 4, NCH):
            sd[c].wait()


@functools.cache
def _sc_gather():
    return pl.kernel(
        _sc_body,
        out_type=jax.ShapeDtypeStruct((LOUT * B, D), jnp.float32),
        mesh=plsc.VectorSubcoreMesh(core_axis_name="c", subcore_axis_name="s",
                                    num_cores=NC, num_subcores=NS),
        scratch_types=[
            pltpu.VMEM((1, L), jnp.int32),
            pltpu.VMEM((NCH, CH), jnp.int32),
            pltpu.VMEM((NCH, CH), jnp.int32),
            pltpu.VMEM((CH, D), jnp.float32),
            pltpu.VMEM((CH, D), jnp.float32),
            pltpu.VMEM((CH, D), jnp.float32),
            pltpu.VMEM((CH, D), jnp.float32),
            pltpu.SemaphoreType.DMA,
            pltpu.SemaphoreType.DMA,
            pltpu.SemaphoreType.DMA,
            pltpu.SemaphoreType.DMA,
            pltpu.SemaphoreType.DMA,
            pltpu.SemaphoreType.DMA,
            pltpu.SemaphoreType.DMA,
        ],
        compiler_params=pltpu.CompilerParams(needs_layout_passes=False),
    )


def kernel(x, noise):
    assert x.shape == (B, LIN, D) and noise.shape == (B, L)
    ranks = _ranks(noise)
    x2 = jnp.transpose(x, (1, 0, 2)).reshape(LIN * B, D)
    out2 = _sc_gather()(x2, ranks)
    return jnp.transpose(out2.reshape(LOUT, B, D), (1, 0, 2))


# speedup vs baseline: 1.0616x; 1.0616x over previous
import functools

import jax
import jax.numpy as jnp
from jax import lax
from jax.experimental import pallas as pl
from jax.experimental.pallas import tpu as pltpu
from jax.experimental.pallas import tpu_sc as plsc

B = 64
L = 1024
D = 768
KEEP = 512
NPREF = 1
LIN = L + NPREF
LOUT = KEEP + NPREF

NC = 2
NS = 16
NW = NC * NS
ROWS_PER_W = B // NW
LANES = 16

CH = 64
NCH = 8


RG = 8


def _rank_body(nlane_ref, out_ref):
    T = 128
    NT = L // T
    tie = (lax.broadcasted_iota(jnp.int32, (T, T), 0)
           < lax.broadcasted_iota(jnp.int32, (T, T), 1))
    for g in range(RG):
        nj = nlane_ref[g]
        nk = jnp.swapaxes(nj, 0, 1)
        for t in range(NT):
            njt = lax.slice(nj, (0, t * T), (1, (t + 1) * T))
            kd = lax.slice(nk, (t * T, 0), ((t + 1) * T, 1))
            pd = (kd < njt) | ((kd == njt) & tie)
            acc = jnp.sum(pd.astype(jnp.int32), axis=0, keepdims=True)
            if t > 0:
                ka = lax.slice(nk, (0, 0), (t * T, 1))
                acc = acc + jnp.sum((ka <= njt).astype(jnp.int32),
                                    axis=0, keepdims=True)
            if t < NT - 1:
                kb = lax.slice(nk, ((t + 1) * T, 0), (L, 1))
                acc = acc + jnp.sum((kb < njt).astype(jnp.int32),
                                    axis=0, keepdims=True)
            out_ref[g, :, pl.ds(t * T, T)] = acc


def _ranks(noise):
    noise3 = noise.reshape(B, 1, L)
    out = pl.pallas_call(
        _rank_body,
        grid=(B // RG,),
        in_specs=[pl.BlockSpec((RG, 1, L), lambda b: (b, 0, 0))],
        out_specs=pl.BlockSpec((RG, 1, L), lambda b: (b, 0, 0)),
        out_shape=jax.ShapeDtypeStruct((B, 1, L), jnp.int32),
    )(noise3)
    return out


def _sc_body(x_hbm, ranks_hbm, out_hbm, rank_v, kp_v, oix_v, buf0, buf1,
             sem0, sem1, semw, semp):
    wid = lax.axis_index("s") * NC + lax.axis_index("c")
    lane = jnp.arange(LANES, dtype=jnp.int32)

    @pl.when(wid == 0)
    def _():
        pltpu.async_copy(x_hbm.at[pl.ds(0, B)], out_hbm.at[pl.ds(0, B)],
                         semp).wait()

    for i in range(ROWS_PER_W):
        b = wid * ROWS_PER_W + i
        pltpu.sync_copy(ranks_hbm.at[b], rank_v)
        for c in range(NCH):
            for g in range(CH // LANES):
                oix_v[c, pl.ds(g * LANES, LANES)] = (
                    lane * B + ((1 + c * CH + g * LANES) * B + b))
        for ci in range(L // LANES):
            r = rank_v[0, pl.ds(ci * LANES, LANES)]
            src = lane * B + ((ci * LANES + 1) * B + b)
            plsc.store_scatter(kp_v, [r // CH, r % CH], src, mask=r < KEEP)
        bufs = (buf0, buf1)
        sems = (sem0, sem1)
        descs = [None] * NCH
        descs[0] = pltpu.async_copy(x_hbm.at[kp_v.at[0]], bufs[0], sems[0])
        descs[1] = pltpu.async_copy(x_hbm.at[kp_v.at[1]], bufs[1], sems[1])
        for c in range(NCH):
            descs[c].wait()
            pltpu.async_copy(bufs[c % 2], out_hbm.at[oix_v.at[c]],
                             semw).wait()
            if c + 2 < NCH:
                descs[c + 2] = pltpu.async_copy(
                    x_hbm.at[kp_v.at[c + 2]], bufs[c % 2], sems[c % 2])


@functools.cache
def _sc_gather():
    return pl.kernel(
        _sc_body,
        out_type=jax.ShapeDtypeStruct((LOUT * B, D), jnp.float32),
        mesh=plsc.VectorSubcoreMesh(core_axis_name="c", subcore_axis_name="s",
                                    num_cores=NC, num_subcores=NS),
        scratch_types=[
            pltpu.VMEM((1, L), jnp.int32),
            pltpu.VMEM((NCH, CH), jnp.int32),
            pltpu.VMEM((NCH, CH), jnp.int32),
            pltpu.VMEM((CH, D), jnp.float32),
            pltpu.VMEM((CH, D), jnp.float32),
            pltpu.SemaphoreType.DMA,
            pltpu.SemaphoreType.DMA,
            pltpu.SemaphoreType.DMA,
            pltpu.SemaphoreType.DMA,
        ],
        compiler_params=pltpu.CompilerParams(needs_layout_passes=False),
    )


def kernel(x, noise):
    assert x.shape == (B, LIN, D) and noise.shape == (B, L)
    ranks = _ranks(noise)
    x2 = jnp.transpose(x, (1, 0, 2)).reshape(LIN * B, D)
    out2 = _sc_gather()(x2, ranks)
    return jnp.transpose(out2.reshape(LOUT, B, D), (1, 0, 2))


# speedup vs baseline: 1.0803x; 1.0176x over previous
import functools

import jax
import jax.numpy as jnp
from jax import lax
from jax.experimental import pallas as pl
from jax.experimental.pallas import tpu as pltpu
from jax.experimental.pallas import tpu_sc as plsc

B = 64
L = 1024
D = 768
KEEP = 512
NPREF = 1
LIN = L + NPREF
LOUT = KEEP + NPREF

NC = 2
NS = 16
NW = NC * NS
ROWS_PER_W = B // NW
LANES = 16

CH = 64
NCH = 8


RG = 16


def _rank_body(nlane_ref, out_ref):
    T = 128
    NT = L // T
    tie = (lax.broadcasted_iota(jnp.int32, (T, T), 0)
           < lax.broadcasted_iota(jnp.int32, (T, T), 1))
    for g in range(RG):
        nj = nlane_ref[pl.ds(g, 1)]
        nk = jnp.swapaxes(nj, 0, 1)
        for t in range(NT):
            njt = lax.slice(nj, (0, t * T), (1, (t + 1) * T))
            kd = lax.slice(nk, (t * T, 0), ((t + 1) * T, 1))
            pd = (kd < njt) | ((kd == njt) & tie)
            acc = jnp.sum(pd.astype(jnp.int32), axis=0, keepdims=True)
            if t > 0:
                ka = lax.slice(nk, (0, 0), (t * T, 1))
                acc = acc + jnp.sum((ka <= njt).astype(jnp.int32),
                                    axis=0, keepdims=True)
            if t < NT - 1:
                kb = lax.slice(nk, ((t + 1) * T, 0), (L, 1))
                acc = acc + jnp.sum((kb < njt).astype(jnp.int32),
                                    axis=0, keepdims=True)
            out_ref[g, :, pl.ds(t * T, T)] = acc


def _ranks(noise):
    out = pl.pallas_call(
        _rank_body,
        grid=(B // RG,),
        in_specs=[pl.BlockSpec((RG, L), lambda b: (b, 0))],
        out_specs=pl.BlockSpec((RG, 1, L), lambda b: (b, 0, 0)),
        out_shape=jax.ShapeDtypeStruct((B, 1, L), jnp.int32),
    )(noise)
    return out


def _sc_body(x_hbm, ranks_hbm, out_hbm, rank_v, kp_v, oix_v, buf0, buf1,
             sem0, sem1, semw, semp):
    wid = lax.axis_index("s") * NC + lax.axis_index("c")
    lane = jnp.arange(LANES, dtype=jnp.int32)

    @pl.when(wid == 0)
    def _():
        pltpu.async_copy(x_hbm.at[pl.ds(0, B)], out_hbm.at[pl.ds(0, B)],
                         semp).wait()

    for i in range(ROWS_PER_W):
        b = wid * ROWS_PER_W + i
        pltpu.sync_copy(ranks_hbm.at[b], rank_v)
        for c in range(NCH):
            for g in range(CH // LANES):
                oix_v[c, pl.ds(g * LANES, LANES)] = (
                    lane * B + ((1 + c * CH + g * LANES) * B + b))
        for ci in range(L // LANES):
            r = rank_v[0, pl.ds(ci * LANES, LANES)]
            src = lane * B + ((ci * LANES + 1) * B + b)
            plsc.store_scatter(kp_v, [r // CH, r % CH], src, mask=r < KEEP)
        bufs = (buf0, buf1)
        sems = (sem0, sem1)
        descs = [None] * NCH
        descs[0] = pltpu.async_copy(x_hbm.at[kp_v.at[0]], bufs[0], sems[0])
        descs[1] = pltpu.async_copy(x_hbm.at[kp_v.at[1]], bufs[1], sems[1])
        for c in range(NCH):
            descs[c].wait()
            pltpu.async_copy(bufs[c % 2], out_hbm.at[oix_v.at[c]],
                             semw).wait()
            if c + 2 < NCH:
                descs[c + 2] = pltpu.async_copy(
                    x_hbm.at[kp_v.at[c + 2]], bufs[c % 2], sems[c % 2])


@functools.cache
def _sc_gather():
    return pl.kernel(
        _sc_body,
        out_type=jax.ShapeDtypeStruct((LOUT * B, D), jnp.float32),
        mesh=plsc.VectorSubcoreMesh(core_axis_name="c", subcore_axis_name="s",
                                    num_cores=NC, num_subcores=NS),
        scratch_types=[
            pltpu.VMEM((1, L), jnp.int32),
            pltpu.VMEM((NCH, CH), jnp.int32),
            pltpu.VMEM((NCH, CH), jnp.int32),
            pltpu.VMEM((CH, D), jnp.float32),
            pltpu.VMEM((CH, D), jnp.float32),
            pltpu.SemaphoreType.DMA,
            pltpu.SemaphoreType.DMA,
            pltpu.SemaphoreType.DMA,
            pltpu.SemaphoreType.DMA,
        ],
        compiler_params=pltpu.CompilerParams(needs_layout_passes=False),
    )


def kernel(x, noise):
    assert x.shape == (B, LIN, D) and noise.shape == (B, L)
    ranks = _ranks(noise)
    x2 = jnp.transpose(x, (1, 0, 2)).reshape(LIN * B, D)
    out2 = _sc_gather()(x2, ranks)
    return jnp.transpose(out2.reshape(LOUT, B, D), (1, 0, 2))


# speedup vs baseline: 1.0860x; 1.0053x over previous
import functools

import jax
import jax.numpy as jnp
from jax import lax
from jax.experimental import pallas as pl
from jax.experimental.pallas import tpu as pltpu
from jax.experimental.pallas import tpu_sc as plsc

B = 64
L = 1024
D = 768
KEEP = 512
NPREF = 1
LIN = L + NPREF
LOUT = KEEP + NPREF

NC = 2
NS = 16
NW = NC * NS
ROWS_PER_W = B // NW
LANES = 16

CH = 64
NCH = 8


RG = 32


def _rank_body(nlane_ref, out_ref):
    T = 128
    NT = L // T
    tie = (lax.broadcasted_iota(jnp.int32, (T, T), 0)
           < lax.broadcasted_iota(jnp.int32, (T, T), 1))
    for g in range(RG):
        nj = nlane_ref[pl.ds(g, 1)]
        nk = jnp.swapaxes(nj, 0, 1)
        for t in range(NT):
            njt = lax.slice(nj, (0, t * T), (1, (t + 1) * T))
            kd = lax.slice(nk, (t * T, 0), ((t + 1) * T, 1))
            pd = (kd < njt) | ((kd == njt) & tie)
            acc = jnp.sum(pd.astype(jnp.int32), axis=0, keepdims=True)
            if t > 0:
                ka = lax.slice(nk, (0, 0), (t * T, 1))
                acc = acc + jnp.sum((ka <= njt).astype(jnp.int32),
                                    axis=0, keepdims=True)
            if t < NT - 1:
                kb = lax.slice(nk, ((t + 1) * T, 0), (L, 1))
                acc = acc + jnp.sum((kb < njt).astype(jnp.int32),
                                    axis=0, keepdims=True)
            out_ref[g, :, pl.ds(t * T, T)] = acc


def _ranks(noise):
    out = pl.pallas_call(
        _rank_body,
        grid=(B // RG,),
        in_specs=[pl.BlockSpec((RG, L), lambda b: (b, 0))],
        out_specs=pl.BlockSpec((RG, 1, L), lambda b: (b, 0, 0)),
        out_shape=jax.ShapeDtypeStruct((B, 1, L), jnp.int32),
    )(noise)
    return out


def _sc_body(x_hbm, ranks_hbm, out_hbm, rank_v, kp_v, oix_v, buf0, buf1,
             sem0, sem1, semw, semp):
    wid = lax.axis_index("s") * NC + lax.axis_index("c")
    lane = jnp.arange(LANES, dtype=jnp.int32)

    @pl.when(wid == 0)
    def _():
        pltpu.async_copy(x_hbm.at[pl.ds(0, B)], out_hbm.at[pl.ds(0, B)],
                         semp).wait()

    for i in range(ROWS_PER_W):
        b = wid * ROWS_PER_W + i
        pltpu.sync_copy(ranks_hbm.at[b], rank_v)
        for c in range(NCH):
            for g in range(CH // LANES):
                oix_v[c, pl.ds(g * LANES, LANES)] = (
                    lane * B + ((1 + c * CH + g * LANES) * B + b))
        for ci in range(L // LANES):
            r = rank_v[0, pl.ds(ci * LANES, LANES)]
            src = lane * B + ((ci * LANES + 1) * B + b)
            plsc.store_scatter(kp_v, [r // CH, r % CH], src, mask=r < KEEP)
        bufs = (buf0, buf1)
        sems = (sem0, sem1)
        descs = [None] * NCH
        descs[0] = pltpu.async_copy(x_hbm.at[kp_v.at[0]], bufs[0], sems[0])
        descs[1] = pltpu.async_copy(x_hbm.at[kp_v.at[1]], bufs[1], sems[1])
        for c in range(NCH):
            descs[c].wait()
            pltpu.async_copy(bufs[c % 2], out_hbm.at[oix_v.at[c]],
                             semw).wait()
            if c + 2 < NCH:
                descs[c + 2] = pltpu.async_copy(
                    x_hbm.at[kp_v.at[c + 2]], bufs[c % 2], sems[c % 2])


@functools.cache
def _sc_gather():
    return pl.kernel(
        _sc_body,
        out_type=jax.ShapeDtypeStruct((LOUT * B, D), jnp.float32),
        mesh=plsc.VectorSubcoreMesh(core_axis_name="c", subcore_axis_name="s",
                                    num_cores=NC, num_subcores=NS),
        scratch_types=[
            pltpu.VMEM((1, L), jnp.int32),
            pltpu.VMEM((NCH, CH), jnp.int32),
            pltpu.VMEM((NCH, CH), jnp.int32),
            pltpu.VMEM((CH, D), jnp.float32),
            pltpu.VMEM((CH, D), jnp.float32),
            pltpu.SemaphoreType.DMA,
            pltpu.SemaphoreType.DMA,
            pltpu.SemaphoreType.DMA,
            pltpu.SemaphoreType.DMA,
        ],
        compiler_params=pltpu.CompilerParams(needs_layout_passes=False),
    )


def kernel(x, noise):
    assert x.shape == (B, LIN, D) and noise.shape == (B, L)
    ranks = _ranks(noise)
    x2 = jnp.transpose(x, (1, 0, 2)).reshape(LIN * B, D)
    out2 = _sc_gather()(x2, ranks)
    return jnp.transpose(out2.reshape(LOUT, B, D), (1, 0, 2))
